# baseline (speedup 1.0000x reference)
import functools

import jax
import jax.numpy as jnp
from jax import lax
from jax.experimental import pallas as pl
from jax.experimental.pallas import tpu as pltpu
from jax.experimental.pallas import tpu_sc as plsc

_D = 64
_W = 128
_NW = 32
_G = 128
_NBUF = 4


@functools.partial(jax.jit, static_argnums=(2,))
def _gather_rows(table, flat_idx, n):
    b_per_w = n // _NW
    n_t = b_per_w // _G
    n_outer = n_t // _NBUF
    mesh = plsc.VectorSubcoreMesh(core_axis_name="c", subcore_axis_name="s")

    @functools.partial(
        pl.kernel,
        mesh=mesh,
        out_type=jax.ShapeDtypeStruct((n, _W), jnp.float32),
        scratch_types=(
            [pltpu.VMEM((b_per_w,), jnp.int32)]
            + [pltpu.VMEM((_G, _W), jnp.float32) for _ in range(_NBUF)]
            + [pltpu.SemaphoreType.DMA for _ in range(2 * _NBUF)]
        ),
        compiler_params=pltpu.CompilerParams(use_tc_tiling_on_sc=False),
    )
    def k(table_hbm, idx_hbm, out_hbm, idx_v, *rest):
        bufs = rest[:_NBUF]
        gsems = rest[_NBUF:2 * _NBUF]
        wsems = rest[2 * _NBUF:]
        wid = lax.axis_index("s") * 2 + lax.axis_index("c")
        base = wid * b_per_w
        pltpu.sync_copy(idx_hbm.at[pl.ds(base, b_per_w)], idx_v)

        def fire_gather(t, b):
            pltpu.async_copy(
                table_hbm.at[idx_v.at[pl.ds(t * _G, _G)]], bufs[b], gsems[b])

        def drain_gather(b):
            pltpu.make_async_copy(
                table_hbm.at[pl.ds(0, _G)], bufs[b], gsems[b]).wait()

        def fire_write(t, b):
            pltpu.async_copy(
                bufs[b], out_hbm.at[pl.ds(base + t * _G, _G)], wsems[b])

        def drain_write(b):
            pltpu.make_async_copy(
                bufs[b], out_hbm.at[pl.ds(base, _G)], wsems[b]).wait()

        for b in range(_NBUF):
            fire_gather(b, b)

        def round_body(g, carry):
            for b in range(_NBUF):
                drain_gather(b)
                fire_write((g - 1) * _NBUF + b, b)
            for b in range(_NBUF):
                drain_write(b)
                fire_gather(g * _NBUF + b, b)
            return carry

        lax.fori_loop(1, n_outer, round_body, 0)

        for b in range(_NBUF):
            drain_gather(b)
            fire_write((n_outer - 1) * _NBUF + b, b)
        for b in range(_NBUF):
            drain_write(b)

    return k(table, flat_idx)


def kernel(inputs, weight):
    b, s = inputs.shape
    n = b * s
    flat_idx = inputs.reshape(n).astype(jnp.int32)
    table = jnp.pad(weight, ((0, 0), (0, _W - _D)))
    out = _gather_rows(table, flat_idx, n)
    return out[:, :_D].reshape(b, s, _D)


# speedup vs baseline: 1.1743x; 1.1743x over previous
import functools

import jax
import jax.numpy as jnp
from jax import lax
from jax.experimental import pallas as pl
from jax.experimental.pallas import tpu as pltpu
from jax.experimental.pallas import tpu_sc as plsc

_D = 64
_W = 128
_NW = 32
_G = 128
_NBUF = 4


@functools.partial(jax.jit, static_argnums=(2,))
def _gather_rows(table2, flat_idx2, n):
    b_per_w = n // _NW
    n_t = b_per_w // _G
    n_outer = n_t // _NBUF
    mesh = plsc.VectorSubcoreMesh(core_axis_name="c", subcore_axis_name="s")

    @functools.partial(
        pl.kernel,
        mesh=mesh,
        out_type=jax.ShapeDtypeStruct((n, _W), jnp.float32),
        scratch_types=(
            [pltpu.VMEM((b_per_w,), jnp.int32)]
            + [pltpu.VMEM((_G, _D), jnp.float32) for _ in range(_NBUF)]
            + [pltpu.SemaphoreType.DMA for _ in range(2 * _NBUF)]
        ),
        compiler_params=pltpu.CompilerParams(use_tc_tiling_on_sc=False),
    )
    def k(table_hbm, idx_hbm, out_hbm, idx_v, *rest):
        bufs = rest[:_NBUF]
        gsems = rest[_NBUF:2 * _NBUF]
        wsems = rest[2 * _NBUF:]
        wid = lax.axis_index("s") * 2 + lax.axis_index("c")
        base = wid * b_per_w
        pltpu.sync_copy(idx_hbm.at[pl.ds(base, b_per_w)], idx_v)

        def fire_gather(t, b):
            pltpu.async_copy(
                table_hbm.at[idx_v.at[pl.ds(t * _G, _G)]], bufs[b], gsems[b])

        def drain_gather(b):
            pltpu.make_async_copy(
                table_hbm.at[pl.ds(0, _G)], bufs[b], gsems[b]).wait()

        def fire_write(t, b):
            pltpu.async_copy(
                bufs[b],
                out_hbm.at[pl.ds(base + t * _G, _G), pl.ds(0, _D)], wsems[b])

        def drain_write(b):
            pltpu.make_async_copy(
                bufs[b],
                out_hbm.at[pl.ds(base, _G), pl.ds(0, _D)], wsems[b]).wait()

        for b in range(_NBUF):
            fire_gather(b, b)

        def round_body(g, carry):
            for b in range(_NBUF):
                drain_gather(b)
                fire_write((g - 1) * _NBUF + b, b)
            for b in range(_NBUF):
                drain_write(b)
                fire_gather(g * _NBUF + b, b)
            return carry

        lax.fori_loop(1, n_outer, round_body, 0)

        for b in range(_NBUF):
            drain_gather(b)
            fire_write((n_outer - 1) * _NBUF + b, b)
        for b in range(_NBUF):
            drain_write(b)

    return k(table2, flat_idx2)


def kernel(inputs, weight):
    b, s = inputs.shape
    n = b * s
    flat_idx2 = inputs.reshape(n).astype(jnp.int32) * 2
    table = jnp.pad(weight, ((0, 0), (0, _W - _D)))
    table2 = table.reshape(2 * weight.shape[0], _D)
    out = _gather_rows(table2, flat_idx2, n)
    return out[:, :_D].reshape(b, s, _D)
